# baseline (speedup 1.0000x reference)
import functools

import jax
import jax.numpy as jnp
from jax import lax
from jax.experimental import pallas as pl
from jax.experimental.pallas import tpu as pltpu
from jax.experimental.pallas import tpu_sc as plsc

VOCAB = 1000000
EMBED_DIM = 64

NC = 2
NS = 16
NW = NC * NS

CHUNK = 512


def _make_lookup(B):
    assert B % (NW * CHUNK) == 0
    b_per_w = B // NW
    n_steps = b_per_w // CHUNK
    mesh = plsc.VectorSubcoreMesh(core_axis_name="c", subcore_axis_name="s")

    @functools.partial(
        pl.kernel,
        mesh=mesh,
        compiler_params=pltpu.CompilerParams(use_tc_tiling_on_sc=False),
        out_type=jax.ShapeDtypeStruct((B, EMBED_DIM), jnp.float32),
        scratch_types=[
            pltpu.SMEM((CHUNK,), jnp.int32),
            pltpu.VMEM((CHUNK,), jnp.int32),
            pltpu.VMEM_SHARED((NS, CHUNK), jnp.int32),
            pltpu.VMEM((CHUNK, EMBED_DIM), jnp.float32),
            pltpu.SemaphoreType.DMA,
            pltpu.SemaphoreType.DMA,
        ],
    )
    def lookup(table_hbm, x_hbm, out_hbm, idx_smem, idx_v, idx_sp, rows0,
               sg0, so0):
        wid = lax.axis_index("s") * NC + lax.axis_index("c")
        sid = lax.axis_index("s")
        base = pl.multiple_of(wid * b_per_w, CHUNK)

        pltpu.sync_copy(x_hbm.at[pl.ds(base, CHUNK)], idx_v)
        pltpu.sync_copy(idx_v, idx_sp.at[sid])
        pltpu.sync_copy(idx_sp.at[sid], idx_smem)

        def chunk_body(c, carry):
            def row(j, carry2):
                idx = idx_smem[j]
                pltpu.async_copy(
                    table_hbm.at[pl.ds(idx, 1)],
                    rows0.at[pl.ds(j, 1)],
                    sg0,
                )
                return carry2

            lax.fori_loop(0, CHUNK, row, 0)
            pltpu.make_async_copy(
                table_hbm.at[pl.ds(0, CHUNK)], rows0, sg0).wait()
            pltpu.async_copy(
                rows0, out_hbm.at[pl.ds(base + c * CHUNK, CHUNK)], so0)
            pltpu.make_async_copy(
                rows0, out_hbm.at[pl.ds(0, CHUNK)], so0).wait()
            return carry

        lax.fori_loop(0, n_steps, chunk_body, 0)

    return lookup


def kernel(x, table):
    B = x.shape[0] * x.shape[1]
    out = _make_lookup(B)(table, x.reshape(B))
    return out.reshape(x.shape[0], x.shape[1], EMBED_DIM)


# speedup vs baseline: 1.0334x; 1.0334x over previous
import functools

import jax
import jax.numpy as jnp
from jax import lax
from jax.experimental import pallas as pl
from jax.experimental.pallas import tpu as pltpu
from jax.experimental.pallas import tpu_sc as plsc

VOCAB = 1000000
EMBED_DIM = 64

NC = 2
NS = 16
NW = NC * NS

IDX_W = 512
CHUNK = 512
K = CHUNK // IDX_W


def _make_lookup(B):
    assert B % (NW * CHUNK) == 0
    b_per_w = B // NW
    n_steps = b_per_w // CHUNK
    assert n_steps >= 4 and n_steps % 2 == 0
    mesh = plsc.VectorSubcoreMesh(core_axis_name="c", subcore_axis_name="s")

    @functools.partial(
        pl.kernel,
        mesh=mesh,
        compiler_params=pltpu.CompilerParams(use_tc_tiling_on_sc=False),
        out_type=jax.ShapeDtypeStruct((B, EMBED_DIM), jnp.float32),
        scratch_types=[
            pltpu.VMEM((b_per_w,), jnp.int32),
            pltpu.VMEM((CHUNK, EMBED_DIM), jnp.float32),
            pltpu.VMEM((CHUNK, EMBED_DIM), jnp.float32),
            pltpu.SemaphoreType.DMA,
            pltpu.SemaphoreType.DMA,
            pltpu.SemaphoreType.DMA,
            pltpu.SemaphoreType.DMA,
        ],
    )
    def lookup(table_hbm, x_hbm, out_hbm, idx_all, rows0, rows1,
               sg0, sg1, so0, so1):
        wid = lax.axis_index("s") * NC + lax.axis_index("c")
        base = pl.multiple_of(wid * b_per_w, CHUNK)
        rows = (rows0, rows1)
        sem_g = (sg0, sg1)
        sem_o = (so0, so1)

        pltpu.sync_copy(x_hbm.at[pl.ds(base, b_per_w)], idx_all)

        def fire(c, s):
            off = pl.multiple_of(c * CHUNK, CHUNK)
            for j in range(K):
                pltpu.async_copy(
                    table_hbm.at[idx_all.at[pl.ds(off + j * IDX_W, IDX_W)]],
                    rows[s].at[pl.ds(j * IDX_W, IDX_W)],
                    sem_g[s],
                )

        def drain_g(s):
            pltpu.make_async_copy(
                table_hbm.at[pl.ds(0, CHUNK)], rows[s], sem_g[s]).wait()

        def start_out(c, s):
            off = pl.multiple_of(base + c * CHUNK, CHUNK)
            pltpu.async_copy(rows[s], out_hbm.at[pl.ds(off, CHUNK)], sem_o[s])

        def wait_out(s):
            pltpu.make_async_copy(
                rows[s], out_hbm.at[pl.ds(0, CHUNK)], sem_o[s]).wait()

        fire(0, 0)
        fire(1, 1)
        drain_g(0)
        start_out(0, 0)

        def pair(k, carry):
            for d in range(2):
                c = 2 * k + 1 + d
                s = (1 + d) % 2
                o = 1 - s
                wait_out(o)
                fire(c + 1, o)
                drain_g(s)
                start_out(c, s)
            return carry

        lax.fori_loop(0, (n_steps - 2) // 2, pair, 0)

        drain_g(1)
        start_out(n_steps - 1, 1)
        wait_out(0)
        wait_out(1)

    return lookup


def kernel(x, table):
    B = x.shape[0] * x.shape[1]
    out = _make_lookup(B)(table, x.reshape(B))
    return out.reshape(x.shape[0], x.shape[1], EMBED_DIM)


# speedup vs baseline: 1.0377x; 1.0042x over previous
import functools

import jax
import jax.numpy as jnp
from jax import lax
from jax.experimental import pallas as pl
from jax.experimental.pallas import tpu as pltpu
from jax.experimental.pallas import tpu_sc as plsc

VOCAB = 1000000
EMBED_DIM = 64

NC = 2
NS = 16
NW = NC * NS

CHUNK = 256
KSLOT = 4


def _make_lookup(B):
    assert B % (NW * CHUNK) == 0
    b_per_w = B // NW
    n_steps = b_per_w // CHUNK
    assert (n_steps - KSLOT) % KSLOT == 0 and n_steps >= 2 * KSLOT
    mesh = plsc.VectorSubcoreMesh(core_axis_name="c", subcore_axis_name="s")

    @functools.partial(
        pl.kernel,
        mesh=mesh,
        compiler_params=pltpu.CompilerParams(use_tc_tiling_on_sc=False),
        out_type=jax.ShapeDtypeStruct((B, EMBED_DIM), jnp.float32),
        scratch_types=[
            pltpu.VMEM((b_per_w,), jnp.int32),
        ]
        + [pltpu.VMEM((CHUNK, EMBED_DIM), jnp.float32)] * KSLOT
        + [pltpu.SemaphoreType.DMA] * (2 * KSLOT),
    )
    def lookup(table_hbm, x_hbm, out_hbm, idx_all, *bufs):
        rows = bufs[:KSLOT]
        sem_g = bufs[KSLOT:2 * KSLOT]
        sem_o = bufs[2 * KSLOT:]
        wid = lax.axis_index("s") * NC + lax.axis_index("c")
        base = pl.multiple_of(wid * b_per_w, CHUNK)

        pltpu.sync_copy(x_hbm.at[pl.ds(base, b_per_w)], idx_all)

        def fire(c, s):
            off = pl.multiple_of(c * CHUNK, CHUNK)
            pltpu.async_copy(
                table_hbm.at[idx_all.at[pl.ds(off, CHUNK)]],
                rows[s],
                sem_g[s],
            )

        def drain_g(s):
            pltpu.make_async_copy(
                table_hbm.at[pl.ds(0, CHUNK)], rows[s], sem_g[s]).wait()

        def start_out(c, s):
            off = pl.multiple_of(base + c * CHUNK, CHUNK)
            pltpu.async_copy(rows[s], out_hbm.at[pl.ds(off, CHUNK)], sem_o[s])

        def wait_out(s):
            pltpu.make_async_copy(
                rows[s], out_hbm.at[pl.ds(0, CHUNK)], sem_o[s]).wait()

        for c in range(KSLOT):
            fire(c, c)
        for c in range(KSLOT - 1):
            drain_g(c)
            start_out(c, c)

        def group(k, carry):
            for d in range(KSLOT):
                c = (KSLOT - 1) + KSLOT * k + d
                s = (KSLOT - 1 + d) % KSLOT
                o = d % KSLOT
                wait_out(o)
                fire_off = pl.multiple_of((c + 1) * CHUNK, CHUNK)
                pltpu.async_copy(
                    table_hbm.at[idx_all.at[pl.ds(fire_off, CHUNK)]],
                    rows[o],
                    sem_g[o],
                )
                drain_g(s)
                out_off = pl.multiple_of(base + c * CHUNK, CHUNK)
                pltpu.async_copy(
                    rows[s], out_hbm.at[pl.ds(out_off, CHUNK)], sem_o[s])
            return carry

        lax.fori_loop(0, (n_steps - KSLOT) // KSLOT, group, 0)

        last = n_steps - 1
        s_last = last % KSLOT
        drain_g(s_last)
        start_out(last, s_last)
        for s in range(KSLOT):
            wait_out(s)

    return lookup


def kernel(x, table):
    B = x.shape[0] * x.shape[1]
    out = _make_lookup(B)(table, x.reshape(B))
    return out.reshape(x.shape[0], x.shape[1], EMBED_DIM)
